# baseline (speedup 1.0000x reference)
import functools

import jax
import jax.numpy as jnp
from jax import lax
from jax.experimental import pallas as pl
from jax.experimental.pallas import tpu as pltpu
from jax.experimental.pallas import tpu_sc as plsc


def _mono_key(x):
    bi = lax.bitcast_convert_type(x, jnp.int32)
    m = lax.shift_right_arithmetic(bi, 31)
    ki = bi ^ (m | jnp.int32(-2147483648))
    return lax.bitcast_convert_type(ki, jnp.uint32)


def _colsq_body(x_ref, o_ref):
    ri = pl.program_id(1)
    x = x_ref[...]
    p = jnp.sum(x * x, axis=0, keepdims=True)

    @pl.when(ri == 0)
    def _():
        o_ref[...] = p

    @pl.when(ri != 0)
    def _():
        o_ref[...] += p


def _colsq(URM):
    U, N = URM.shape
    BC = min(512, N)
    BR = min(1024, U)
    grid = (N // BC, U // BR)
    return pl.pallas_call(
        _colsq_body,
        grid=grid,
        in_specs=[pl.BlockSpec((BR, BC), lambda ci, ri: (ri, ci))],
        out_specs=pl.BlockSpec((1, BC), lambda ci, ri: (0, ci)),
        out_shape=jax.ShapeDtypeStruct((1, N), jnp.float32),
        compiler_params=pltpu.CompilerParams(
            dimension_semantics=("parallel", "arbitrary")),
    )(URM)


def _gram_body(xm_ref, xn_ref, n2m_ref, n2n_ref, o_ref):
    kk = pl.program_id(2)
    invm = 1.0 / jnp.maximum(jnp.sqrt(n2m_ref[...]), 1e-12)
    invn = 1.0 / jnp.maximum(jnp.sqrt(n2n_ref[...]), 1e-12)
    a = xm_ref[...] * invm
    b = xn_ref[...] * invn
    p = lax.dot_general(a, b, (((0,), (0,)), ((), ())),
                        preferred_element_type=jnp.float32)

    @pl.when(kk == 0)
    def _():
        o_ref[...] = p

    @pl.when(kk != 0)
    def _():
        o_ref[...] += p


def _gram_sym_body(xm_ref, xn_ref, n2m_ref, n2n_ref, o_ref):
    kk = pl.program_id(1)
    invm = 1.0 / jnp.maximum(jnp.sqrt(n2m_ref[...]), 1e-12)
    invn = 1.0 / jnp.maximum(jnp.sqrt(n2n_ref[...]), 1e-12)
    a = xm_ref[...] * invm
    b = xn_ref[...] * invn
    p = lax.dot_general(a, b, (((0,), (0,)), ((), ())),
                        preferred_element_type=jnp.float32)
    p = jnp.reshape(p, (1,) + p.shape)

    @pl.when(kk == 0)
    def _():
        o_ref[...] = p

    @pl.when(kk != 0)
    def _():
        o_ref[...] += p


def _mirror_body(p_ref, o_ref):
    r = pl.program_id(0)
    c = pl.program_id(1)
    x = p_ref[0]
    xT = jnp.swapaxes(x, 0, 1)
    o_ref[...] = jnp.where(r > c, xT, x)


def _gram(URM, n2):
    U, N = URM.shape
    BM = min(1024, N)
    BK = min(2048, U)
    nb = N // BM
    npairs = nb * (nb + 1) // 2

    def _pi(p):
        tn = 2 * nb + 1
        s = jnp.sqrt(jnp.float32(tn * tn) - 8.0 * p.astype(jnp.float32))
        return jnp.floor((tn - s) * 0.5).astype(p.dtype)

    def _pj(p):
        i = _pi(p)
        return p - (i * nb - (i * (i - 1)) // 2) + i

    def _tri(r, c):
        i = jnp.minimum(r, c)
        j = jnp.maximum(r, c)
        return i * nb - (i * (i - 1)) // 2 + (j - i)

    packed = pl.pallas_call(
        _gram_sym_body,
        grid=(npairs, U // BK),
        in_specs=[
            pl.BlockSpec((BK, BM), lambda p, k: (k, _pi(p))),
            pl.BlockSpec((BK, BM), lambda p, k: (k, _pj(p))),
            pl.BlockSpec((1, BM), lambda p, k: (0, _pi(p))),
            pl.BlockSpec((1, BM), lambda p, k: (0, _pj(p))),
        ],
        out_specs=pl.BlockSpec((1, BM, BM), lambda p, k: (p, 0, 0)),
        out_shape=jax.ShapeDtypeStruct((npairs, BM, BM), jnp.float32),
        compiler_params=pltpu.CompilerParams(
            dimension_semantics=("parallel", "arbitrary")),
    )(URM, URM, n2, n2)

    return pl.pallas_call(
        _mirror_body,
        grid=(nb, nb),
        in_specs=[pl.BlockSpec((1, BM, BM), lambda r, c: (_tri(r, c), 0, 0))],
        out_specs=pl.BlockSpec((BM, BM), lambda r, c: (r, c)),
        out_shape=jax.ShapeDtypeStruct((N, N), jnp.float32),
        compiler_params=pltpu.CompilerParams(
            dimension_semantics=("parallel", "parallel")),
    )(packed)


def _select_body(k_ref, s_ref, t_ref, jc_ref, *, R, N, idx_bits):
    i = pl.program_id(0)
    kf = k_ref[0, 0].astype(jnp.float32)
    s = s_ref[...]
    keys = _mono_key(s)
    rowg = i * R + lax.broadcasted_iota(jnp.int32, (R, N), 0)
    colg = lax.broadcasted_iota(jnp.int32, (R, N), 1)
    keys = jnp.where(colg == rowg, jnp.uint32(0), keys)

    def rowcount(mask):
        return jnp.sum(mask.astype(jnp.float32), axis=1, keepdims=True)

    t = jnp.full((R, 1), jnp.uint32(1 << 31))
    cnt_ge = jnp.full((R, 1), jnp.float32(N - 1))
    for b in range(29, -1, -1):
        cand = t | jnp.uint32(1 << b)
        cnt = rowcount(keys >= cand)
        take = cnt >= kf
        t = jnp.where(take, cand, t)
        cnt_ge = jnp.where(take, cnt, cnt_ge)

    eq = keys == t
    r = kf - (cnt_ge - rowcount(eq))
    colv = jnp.where(eq, colg, jnp.int32(N))
    lo = jnp.zeros((R, 1), jnp.int32)
    for b in range(idx_bits - 1, -1, -1):
        cand = lo + ((1 << b) - 1)
        f = rowcount(colv <= cand)
        lo = jnp.where(f < r, lo + (1 << b), lo)

    t_ref[...] = t
    jc_ref[...] = lo


def _select(sim, k):
    N = sim.shape[0]
    R = min(512, N)
    idx_bits = max(1, (N - 1).bit_length())
    k_arr = jnp.reshape(jnp.asarray(k, jnp.int32), (1, 1))
    body = functools.partial(_select_body, R=R, N=N, idx_bits=idx_bits)
    return pl.pallas_call(
        body,
        grid=(N // R,),
        in_specs=[
            pl.BlockSpec((1, 1), lambda i: (0, 0)),
            pl.BlockSpec((R, N), lambda i: (i, 0)),
        ],
        out_specs=[
            pl.BlockSpec((R, 1), lambda i: (i, 0)),
            pl.BlockSpec((R, 1), lambda i: (i, 0)),
        ],
        out_shape=[
            jax.ShapeDtypeStruct((N, 1), jnp.uint32),
            jax.ShapeDtypeStruct((N, 1), jnp.int32),
        ],
        compiler_params=pltpu.CompilerParams(
            dimension_semantics=("arbitrary",)),
    )(k_arr, sim)


def _gather_rows(URM, user_ids):
    U, N = URM.shape
    B = user_ids.shape[0]
    info = plsc.get_sparse_core_info()
    NC, NS = info.num_cores, info.num_subcores
    NW = NC * NS
    b_per_w = B // NW
    CH = min(8, b_per_w)
    mesh = plsc.VectorSubcoreMesh(core_axis_name="c", subcore_axis_name="s")

    @functools.partial(
        pl.kernel,
        mesh=mesh,
        out_type=jax.ShapeDtypeStruct((B, N), jnp.float32),
        scratch_types=[
            pltpu.VMEM((b_per_w,), jnp.int32),
            pltpu.VMEM((CH, N), jnp.float32),
            pltpu.SemaphoreType.DMA,
        ],
    )
    def gather(table_hbm, idx_hbm, out_hbm, idx_v, rows_v, sem):
        wid = lax.axis_index("s") * NC + lax.axis_index("c")
        base = wid * b_per_w
        pltpu.sync_copy(idx_hbm.at[pl.ds(base, b_per_w)], idx_v)
        for c in range(b_per_w // CH):
            pltpu.async_copy(
                table_hbm.at[idx_v.at[pl.ds(c * CH, CH)]], rows_v, sem).wait()
            pltpu.sync_copy(rows_v, out_hbm.at[pl.ds(base + c * CH, CH)])

    return gather(URM, user_ids)


def _score_body(u_ref, s_ref, nz_ref, t_ref, jc_ref, o_ref, *, BK, BN):
    jj = pl.program_id(0)
    kk = pl.program_id(1)
    s = s_ref[...]
    keys = _mono_key(s)
    t = t_ref[...]
    jc = jc_ref[...]
    colg = jj * BN + lax.broadcasted_iota(jnp.int32, (BK, BN), 1)
    rowg = kk * BK + lax.broadcasted_iota(jnp.int32, (BK, BN), 0)
    mask = (keys > t) | ((keys == t) & (colg <= jc))
    mask = mask & (colg != rowg)
    w = jnp.where(mask, s, 0.0) + nz_ref[...]
    p = jnp.dot(u_ref[...], w, preferred_element_type=jnp.float32)

    @pl.when(kk == 0)
    def _():
        o_ref[...] = p

    @pl.when(kk != 0)
    def _():
        o_ref[...] += p


def _score(U_sel, sim, noise, t, jc):
    B, N = U_sel.shape
    BK = min(1024, N)
    BN = min(1024, N)
    body = functools.partial(_score_body, BK=BK, BN=BN)
    grid = (N // BN, N // BK)
    return pl.pallas_call(
        body,
        grid=grid,
        in_specs=[
            pl.BlockSpec((B, BK), lambda j, k: (0, k)),
            pl.BlockSpec((BK, BN), lambda j, k: (k, j)),
            pl.BlockSpec((BK, BN), lambda j, k: (k, j)),
            pl.BlockSpec((BK, 1), lambda j, k: (k, 0)),
            pl.BlockSpec((BK, 1), lambda j, k: (k, 0)),
        ],
        out_specs=pl.BlockSpec((B, BN), lambda j, k: (0, j)),
        out_shape=jax.ShapeDtypeStruct((B, N), jnp.float32),
        compiler_params=pltpu.CompilerParams(
            dimension_semantics=("parallel", "arbitrary")),
    )(U_sel, sim, noise, t, jc)


def kernel(URM, noise, user_ids, topk):
    n2 = _colsq(URM)
    sim = _gram(URM, n2)
    t, jc = _select(sim, topk)
    U_sel = _gather_rows(URM, user_ids.astype(jnp.int32))
    out = _score(U_sel, sim, noise, t, jc)
    return out.astype(jnp.float32)


# speedup vs baseline: 1.1038x; 1.1038x over previous
import functools

import jax
import jax.numpy as jnp
from jax import lax
from jax.experimental import pallas as pl
from jax.experimental.pallas import tpu as pltpu
from jax.experimental.pallas import tpu_sc as plsc


def _mono_key(x):
    bi = lax.bitcast_convert_type(x, jnp.int32)
    m = lax.shift_right_arithmetic(bi, 31)
    ki = bi ^ (m | jnp.int32(-2147483648))
    return lax.bitcast_convert_type(ki, jnp.uint32)


def _colsq_body(x_ref, o_ref):
    ri = pl.program_id(1)
    x = x_ref[...]
    p = jnp.sum(x * x, axis=0, keepdims=True)

    @pl.when(ri == 0)
    def _():
        o_ref[...] = p

    @pl.when(ri != 0)
    def _():
        o_ref[...] += p


def _colsq(URM):
    U, N = URM.shape
    BC = min(512, N)
    BR = min(1024, U)
    grid = (N // BC, U // BR)
    return pl.pallas_call(
        _colsq_body,
        grid=grid,
        in_specs=[pl.BlockSpec((BR, BC), lambda ci, ri: (ri, ci))],
        out_specs=pl.BlockSpec((1, BC), lambda ci, ri: (0, ci)),
        out_shape=jax.ShapeDtypeStruct((1, N), jnp.float32),
        compiler_params=pltpu.CompilerParams(
            dimension_semantics=("parallel", "arbitrary")),
    )(URM)


def _gram_body(xm_ref, xn_ref, n2m_ref, n2n_ref, o_ref):
    kk = pl.program_id(2)
    invm = 1.0 / jnp.maximum(jnp.sqrt(n2m_ref[...]), 1e-12)
    invn = 1.0 / jnp.maximum(jnp.sqrt(n2n_ref[...]), 1e-12)
    a = xm_ref[...] * invm
    b = xn_ref[...] * invn
    p = lax.dot_general(a, b, (((0,), (0,)), ((), ())),
                        preferred_element_type=jnp.float32)

    @pl.when(kk == 0)
    def _():
        o_ref[...] = p

    @pl.when(kk != 0)
    def _():
        o_ref[...] += p


def _gram_sym_body(xm_ref, xn_ref, n2m_ref, n2n_ref, o_ref):
    kk = pl.program_id(1)
    invm = 1.0 / jnp.maximum(jnp.sqrt(n2m_ref[...]), 1e-12)
    invn = 1.0 / jnp.maximum(jnp.sqrt(n2n_ref[...]), 1e-12)
    a = xm_ref[...] * invm
    b = xn_ref[...] * invn
    p = lax.dot_general(a, b, (((0,), (0,)), ((), ())),
                        preferred_element_type=jnp.float32)
    p = jnp.reshape(p, (1,) + p.shape)

    @pl.when(kk == 0)
    def _():
        o_ref[...] = p

    @pl.when(kk != 0)
    def _():
        o_ref[...] += p


def _mirror_body(p_ref, o_ref):
    r = pl.program_id(0)
    c = pl.program_id(1)
    x = p_ref[0]
    xT = jnp.swapaxes(x, 0, 1)
    o_ref[...] = jnp.where(r > c, xT, x)


def _gram(URM, n2):
    U, N = URM.shape
    BM = min(1024, N)
    BK = min(2048, U)
    nb = N // BM
    npairs = nb * (nb + 1) // 2

    def _pi(p):
        tn = 2 * nb + 1
        s = jnp.sqrt(jnp.float32(tn * tn) - 8.0 * p.astype(jnp.float32))
        return jnp.floor((tn - s) * 0.5).astype(p.dtype)

    def _pj(p):
        i = _pi(p)
        return p - (i * nb - (i * (i - 1)) // 2) + i

    def _tri(r, c):
        i = jnp.minimum(r, c)
        j = jnp.maximum(r, c)
        return i * nb - (i * (i - 1)) // 2 + (j - i)

    packed = pl.pallas_call(
        _gram_sym_body,
        grid=(npairs, U // BK),
        in_specs=[
            pl.BlockSpec((BK, BM), lambda p, k: (k, _pi(p))),
            pl.BlockSpec((BK, BM), lambda p, k: (k, _pj(p))),
            pl.BlockSpec((1, BM), lambda p, k: (0, _pi(p))),
            pl.BlockSpec((1, BM), lambda p, k: (0, _pj(p))),
        ],
        out_specs=pl.BlockSpec((1, BM, BM), lambda p, k: (p, 0, 0)),
        out_shape=jax.ShapeDtypeStruct((npairs, BM, BM), jnp.float32),
        compiler_params=pltpu.CompilerParams(
            dimension_semantics=("parallel", "arbitrary")),
    )(URM, URM, n2, n2)

    return pl.pallas_call(
        _mirror_body,
        grid=(nb, nb),
        in_specs=[pl.BlockSpec((1, BM, BM), lambda r, c: (_tri(r, c), 0, 0))],
        out_specs=pl.BlockSpec((BM, BM), lambda r, c: (r, c)),
        out_shape=jax.ShapeDtypeStruct((N, N), jnp.float32),
        compiler_params=pltpu.CompilerParams(
            dimension_semantics=("parallel", "parallel")),
    )(packed)


def _select_body(k_ref, s_ref, t_ref, jc_ref, *, R, N, idx_bits):
    i = pl.program_id(0)
    kf = k_ref[0, 0].astype(jnp.float32)
    s = s_ref[...]
    keys = _mono_key(s)
    rowg = i * R + lax.broadcasted_iota(jnp.int32, (R, N), 0)
    colg = lax.broadcasted_iota(jnp.int32, (R, N), 1)
    keys = jnp.where(colg == rowg, jnp.uint32(0), keys)

    def rowcount(mask):
        return jnp.sum(mask.astype(jnp.float32), axis=1, keepdims=True)

    t = jnp.full((R, 1), jnp.uint32(1 << 31))
    cnt_ge = jnp.full((R, 1), jnp.float32(N - 1))
    for b in range(29, -1, -1):
        cand = t | jnp.uint32(1 << b)
        cnt = rowcount(keys >= cand)
        take = cnt >= kf
        t = jnp.where(take, cand, t)
        cnt_ge = jnp.where(take, cnt, cnt_ge)

    eq = keys == t
    r = kf - (cnt_ge - rowcount(eq))
    colv = jnp.where(eq, colg, jnp.int32(N))
    lo = jnp.zeros((R, 1), jnp.int32)
    for b in range(idx_bits - 1, -1, -1):
        cand = lo + ((1 << b) - 1)
        f = rowcount(colv <= cand)
        lo = jnp.where(f < r, lo + (1 << b), lo)

    t_ref[...] = t
    jc_ref[...] = lo


def _select(sim, k):
    N = sim.shape[0]
    R = min(128, N)
    idx_bits = max(1, (N - 1).bit_length())
    k_arr = jnp.reshape(jnp.asarray(k, jnp.int32), (1, 1))
    body = functools.partial(_select_body, R=R, N=N, idx_bits=idx_bits)
    return pl.pallas_call(
        body,
        grid=(N // R,),
        in_specs=[
            pl.BlockSpec((1, 1), lambda i: (0, 0)),
            pl.BlockSpec((R, N), lambda i: (i, 0)),
        ],
        out_specs=[
            pl.BlockSpec((R, 1), lambda i: (i, 0)),
            pl.BlockSpec((R, 1), lambda i: (i, 0)),
        ],
        out_shape=[
            jax.ShapeDtypeStruct((N, 1), jnp.uint32),
            jax.ShapeDtypeStruct((N, 1), jnp.int32),
        ],
        compiler_params=pltpu.CompilerParams(
            dimension_semantics=("arbitrary",)),
    )(k_arr, sim)


def _gather_rows(URM, user_ids):
    U, N = URM.shape
    B = user_ids.shape[0]
    info = plsc.get_sparse_core_info()
    NC, NS = info.num_cores, info.num_subcores
    NW = NC * NS
    b_per_w = B // NW
    CH = min(8, b_per_w)
    mesh = plsc.VectorSubcoreMesh(core_axis_name="c", subcore_axis_name="s")

    @functools.partial(
        pl.kernel,
        mesh=mesh,
        out_type=jax.ShapeDtypeStruct((B, N), jnp.float32),
        scratch_types=[
            pltpu.VMEM((b_per_w,), jnp.int32),
            pltpu.VMEM((CH, N), jnp.float32),
            pltpu.SemaphoreType.DMA,
        ],
    )
    def gather(table_hbm, idx_hbm, out_hbm, idx_v, rows_v, sem):
        wid = lax.axis_index("s") * NC + lax.axis_index("c")
        base = wid * b_per_w
        pltpu.sync_copy(idx_hbm.at[pl.ds(base, b_per_w)], idx_v)
        for c in range(b_per_w // CH):
            pltpu.async_copy(
                table_hbm.at[idx_v.at[pl.ds(c * CH, CH)]], rows_v, sem).wait()
            pltpu.sync_copy(rows_v, out_hbm.at[pl.ds(base + c * CH, CH)])

    return gather(URM, user_ids)


def _score_body(u_ref, s_ref, nz_ref, t_ref, jc_ref, o_ref, *, BK, BN):
    jj = pl.program_id(0)
    kk = pl.program_id(1)
    s = s_ref[...]
    keys = _mono_key(s)
    t = t_ref[...]
    jc = jc_ref[...]
    colg = jj * BN + lax.broadcasted_iota(jnp.int32, (BK, BN), 1)
    rowg = kk * BK + lax.broadcasted_iota(jnp.int32, (BK, BN), 0)
    mask = (keys > t) | ((keys == t) & (colg <= jc))
    mask = mask & (colg != rowg)
    w = jnp.where(mask, s, 0.0) + nz_ref[...]
    p = jnp.dot(u_ref[...], w, preferred_element_type=jnp.float32)

    @pl.when(kk == 0)
    def _():
        o_ref[...] = p

    @pl.when(kk != 0)
    def _():
        o_ref[...] += p


def _score(U_sel, sim, noise, t, jc):
    B, N = U_sel.shape
    BK = min(1024, N)
    BN = min(1024, N)
    body = functools.partial(_score_body, BK=BK, BN=BN)
    grid = (N // BN, N // BK)
    return pl.pallas_call(
        body,
        grid=grid,
        in_specs=[
            pl.BlockSpec((B, BK), lambda j, k: (0, k)),
            pl.BlockSpec((BK, BN), lambda j, k: (k, j)),
            pl.BlockSpec((BK, BN), lambda j, k: (k, j)),
            pl.BlockSpec((BK, 1), lambda j, k: (k, 0)),
            pl.BlockSpec((BK, 1), lambda j, k: (k, 0)),
        ],
        out_specs=pl.BlockSpec((B, BN), lambda j, k: (0, j)),
        out_shape=jax.ShapeDtypeStruct((B, N), jnp.float32),
        compiler_params=pltpu.CompilerParams(
            dimension_semantics=("parallel", "arbitrary")),
    )(U_sel, sim, noise, t, jc)


def kernel(URM, noise, user_ids, topk):
    n2 = _colsq(URM)
    sim = _gram(URM, n2)
    t, jc = _select(sim, topk)
    U_sel = _gather_rows(URM, user_ids.astype(jnp.int32))
    out = _score(U_sel, sim, noise, t, jc)
    return out.astype(jnp.float32)


# speedup vs baseline: 1.1261x; 1.0202x over previous
import functools

import jax
import jax.numpy as jnp
from jax import lax
from jax.experimental import pallas as pl
from jax.experimental.pallas import tpu as pltpu
from jax.experimental.pallas import tpu_sc as plsc


def _mono_key(x):
    bi = lax.bitcast_convert_type(x, jnp.int32)
    m = lax.shift_right_arithmetic(bi, 31)
    ki = bi ^ (m | jnp.int32(-2147483648))
    return lax.bitcast_convert_type(ki, jnp.uint32)


def _colsq_body(x_ref, o_ref):
    ri = pl.program_id(1)
    x = x_ref[...]
    p = jnp.sum(x * x, axis=0, keepdims=True)

    @pl.when(ri == 0)
    def _():
        o_ref[...] = p

    @pl.when(ri != 0)
    def _():
        o_ref[...] += p


def _colsq(URM):
    U, N = URM.shape
    BC = min(512, N)
    BR = min(1024, U)
    grid = (N // BC, U // BR)
    return pl.pallas_call(
        _colsq_body,
        grid=grid,
        in_specs=[pl.BlockSpec((BR, BC), lambda ci, ri: (ri, ci))],
        out_specs=pl.BlockSpec((1, BC), lambda ci, ri: (0, ci)),
        out_shape=jax.ShapeDtypeStruct((1, N), jnp.float32),
        compiler_params=pltpu.CompilerParams(
            dimension_semantics=("parallel", "arbitrary")),
    )(URM)


def _gram_body(xm_ref, xn_ref, n2m_ref, n2n_ref, o_ref):
    kk = pl.program_id(2)
    invm = 1.0 / jnp.maximum(jnp.sqrt(n2m_ref[...]), 1e-12)
    invn = 1.0 / jnp.maximum(jnp.sqrt(n2n_ref[...]), 1e-12)
    a = xm_ref[...] * invm
    b = xn_ref[...] * invn
    p = lax.dot_general(a, b, (((0,), (0,)), ((), ())),
                        preferred_element_type=jnp.float32)

    @pl.when(kk == 0)
    def _():
        o_ref[...] = p

    @pl.when(kk != 0)
    def _():
        o_ref[...] += p


def _gram_sym_body(xm_ref, xn_ref, n2m_ref, n2n_ref, o_ref):
    kk = pl.program_id(1)
    invm = 1.0 / jnp.maximum(jnp.sqrt(n2m_ref[...]), 1e-12)
    invn = 1.0 / jnp.maximum(jnp.sqrt(n2n_ref[...]), 1e-12)
    a = xm_ref[...] * invm
    b = xn_ref[...] * invn
    p = lax.dot_general(a, b, (((0,), (0,)), ((), ())),
                        preferred_element_type=jnp.float32)
    p = jnp.reshape(p, (1,) + p.shape)

    @pl.when(kk == 0)
    def _():
        o_ref[...] = p

    @pl.when(kk != 0)
    def _():
        o_ref[...] += p


def _mirror_body(p_ref, o_ref):
    r = pl.program_id(0)
    c = pl.program_id(1)
    x = p_ref[0]
    xT = jnp.swapaxes(x, 0, 1)
    o_ref[...] = jnp.where(r > c, xT, x)


def _gram(URM, n2):
    U, N = URM.shape
    BM = min(1024, N)
    BK = min(2048, U)
    nb = N // BM
    npairs = nb * (nb + 1) // 2

    def _pi(p):
        tn = 2 * nb + 1
        s = jnp.sqrt(jnp.float32(tn * tn) - 8.0 * p.astype(jnp.float32))
        return jnp.floor((tn - s) * 0.5).astype(p.dtype)

    def _pj(p):
        i = _pi(p)
        return p - (i * nb - (i * (i - 1)) // 2) + i

    def _tri(r, c):
        i = jnp.minimum(r, c)
        j = jnp.maximum(r, c)
        return i * nb - (i * (i - 1)) // 2 + (j - i)

    packed = pl.pallas_call(
        _gram_sym_body,
        grid=(npairs, U // BK),
        in_specs=[
            pl.BlockSpec((BK, BM), lambda p, k: (k, _pi(p))),
            pl.BlockSpec((BK, BM), lambda p, k: (k, _pj(p))),
            pl.BlockSpec((1, BM), lambda p, k: (0, _pi(p))),
            pl.BlockSpec((1, BM), lambda p, k: (0, _pj(p))),
        ],
        out_specs=pl.BlockSpec((1, BM, BM), lambda p, k: (p, 0, 0)),
        out_shape=jax.ShapeDtypeStruct((npairs, BM, BM), jnp.float32),
        compiler_params=pltpu.CompilerParams(
            dimension_semantics=("parallel", "arbitrary")),
    )(URM, URM, n2, n2)

    return pl.pallas_call(
        _mirror_body,
        grid=(nb, nb),
        in_specs=[pl.BlockSpec((1, BM, BM), lambda r, c: (_tri(r, c), 0, 0))],
        out_specs=pl.BlockSpec((BM, BM), lambda r, c: (r, c)),
        out_shape=jax.ShapeDtypeStruct((N, N), jnp.float32),
        compiler_params=pltpu.CompilerParams(
            dimension_semantics=("parallel", "parallel")),
    )(packed)


def _select_body(k_ref, s_ref, t_ref, jc_ref, *, R, N, idx_bits):
    i = pl.program_id(0)
    kf = k_ref[0, 0].astype(jnp.float32)
    s = s_ref[...]
    keys = _mono_key(s)
    rowg = i * R + lax.broadcasted_iota(jnp.int32, (R, N), 0)
    colg = lax.broadcasted_iota(jnp.int32, (R, N), 1)
    keys = jnp.where(colg == rowg, jnp.uint32(0), keys)

    def rowcount(mask):
        return jnp.sum(mask.astype(jnp.float32), axis=1, keepdims=True)

    t = jnp.full((R, 1), jnp.uint32(1 << 31))
    cnt_ge = jnp.full((R, 1), jnp.float32(N - 1))
    for b in range(29, -1, -1):
        cand = t | jnp.uint32(1 << b)
        cnt = rowcount(keys >= cand)
        take = cnt >= kf
        t = jnp.where(take, cand, t)
        cnt_ge = jnp.where(take, cnt, cnt_ge)

    eq = keys == t
    r = kf - (cnt_ge - rowcount(eq))
    colv = jnp.where(eq, colg, jnp.int32(N))
    lo = jnp.zeros((R, 1), jnp.int32)
    for b in range(idx_bits - 1, -1, -1):
        cand = lo + ((1 << b) - 1)
        f = rowcount(colv <= cand)
        lo = jnp.where(f < r, lo + (1 << b), lo)

    t_ref[...] = t
    jc_ref[...] = lo


def _select(sim, k):
    N = sim.shape[0]
    R = min(256, N)
    idx_bits = max(1, (N - 1).bit_length())
    k_arr = jnp.reshape(jnp.asarray(k, jnp.int32), (1, 1))
    body = functools.partial(_select_body, R=R, N=N, idx_bits=idx_bits)
    return pl.pallas_call(
        body,
        grid=(N // R,),
        in_specs=[
            pl.BlockSpec((1, 1), lambda i: (0, 0)),
            pl.BlockSpec((R, N), lambda i: (i, 0)),
        ],
        out_specs=[
            pl.BlockSpec((R, 1), lambda i: (i, 0)),
            pl.BlockSpec((R, 1), lambda i: (i, 0)),
        ],
        out_shape=[
            jax.ShapeDtypeStruct((N, 1), jnp.uint32),
            jax.ShapeDtypeStruct((N, 1), jnp.int32),
        ],
        compiler_params=pltpu.CompilerParams(
            dimension_semantics=("arbitrary",)),
    )(k_arr, sim)


def _gather_rows(URM, user_ids):
    U, N = URM.shape
    B = user_ids.shape[0]
    info = plsc.get_sparse_core_info()
    NC, NS = info.num_cores, info.num_subcores
    NW = NC * NS
    b_per_w = B // NW
    CH = min(8, b_per_w)
    mesh = plsc.VectorSubcoreMesh(core_axis_name="c", subcore_axis_name="s")

    @functools.partial(
        pl.kernel,
        mesh=mesh,
        out_type=jax.ShapeDtypeStruct((B, N), jnp.float32),
        scratch_types=[
            pltpu.VMEM((b_per_w,), jnp.int32),
            pltpu.VMEM((CH, N), jnp.float32),
            pltpu.SemaphoreType.DMA,
        ],
    )
    def gather(table_hbm, idx_hbm, out_hbm, idx_v, rows_v, sem):
        wid = lax.axis_index("s") * NC + lax.axis_index("c")
        base = wid * b_per_w
        pltpu.sync_copy(idx_hbm.at[pl.ds(base, b_per_w)], idx_v)
        for c in range(b_per_w // CH):
            pltpu.async_copy(
                table_hbm.at[idx_v.at[pl.ds(c * CH, CH)]], rows_v, sem).wait()
            pltpu.sync_copy(rows_v, out_hbm.at[pl.ds(base + c * CH, CH)])

    return gather(URM, user_ids)


def _score_body(u_ref, s_ref, nz_ref, t_ref, jc_ref, o_ref, *, BK, BN):
    jj = pl.program_id(0)
    kk = pl.program_id(1)
    s = s_ref[...]
    keys = _mono_key(s)
    t = t_ref[...]
    jc = jc_ref[...]
    colg = jj * BN + lax.broadcasted_iota(jnp.int32, (BK, BN), 1)
    rowg = kk * BK + lax.broadcasted_iota(jnp.int32, (BK, BN), 0)
    mask = (keys > t) | ((keys == t) & (colg <= jc))
    mask = mask & (colg != rowg)
    w = jnp.where(mask, s, 0.0) + nz_ref[...]
    p = jnp.dot(u_ref[...], w, preferred_element_type=jnp.float32)

    @pl.when(kk == 0)
    def _():
        o_ref[...] = p

    @pl.when(kk != 0)
    def _():
        o_ref[...] += p


def _score(U_sel, sim, noise, t, jc):
    B, N = U_sel.shape
    BK = min(1024, N)
    BN = min(1024, N)
    body = functools.partial(_score_body, BK=BK, BN=BN)
    grid = (N // BN, N // BK)
    return pl.pallas_call(
        body,
        grid=grid,
        in_specs=[
            pl.BlockSpec((B, BK), lambda j, k: (0, k)),
            pl.BlockSpec((BK, BN), lambda j, k: (k, j)),
            pl.BlockSpec((BK, BN), lambda j, k: (k, j)),
            pl.BlockSpec((BK, 1), lambda j, k: (k, 0)),
            pl.BlockSpec((BK, 1), lambda j, k: (k, 0)),
        ],
        out_specs=pl.BlockSpec((B, BN), lambda j, k: (0, j)),
        out_shape=jax.ShapeDtypeStruct((B, N), jnp.float32),
        compiler_params=pltpu.CompilerParams(
            dimension_semantics=("parallel", "arbitrary")),
    )(U_sel, sim, noise, t, jc)


def kernel(URM, noise, user_ids, topk):
    n2 = _colsq(URM)
    sim = _gram(URM, n2)
    t, jc = _select(sim, topk)
    U_sel = _gather_rows(URM, user_ids.astype(jnp.int32))
    out = _score(U_sel, sim, noise, t, jc)
    return out.astype(jnp.float32)
